# baseline (speedup 1.0000x reference)
import functools

import jax
import jax.numpy as jnp
from jax import lax
from jax.experimental import pallas as pl
from jax.experimental.pallas import tpu as pltpu
from jax.experimental.pallas import tpu_sc as plsc

NC = 2
NS = 16
NW = NC * NS

CHUNK = 128
DPAD = 384

PAD_BLK = 1000
SLICE_PAGES = 8


def _pad_table(W, dim):
    v = W.shape[0]

    def body(w_ref, o_ref):
        o_ref[:, :dim] = w_ref[...]
        o_ref[:, dim:] = jnp.zeros_like(o_ref[:, dim:])

    return pl.pallas_call(
        body,
        grid=(v // PAD_BLK,),
        in_specs=[pl.BlockSpec((PAD_BLK, dim), lambda i: (i, 0))],
        out_specs=pl.BlockSpec((PAD_BLK, DPAD), lambda i: (i, 0)),
        out_shape=jax.ShapeDtypeStruct((v, DPAD), jnp.float32),
    )(W)


def _slice_out(o_pad, b, l, dim):
    def body(i_ref, o_ref):
        o_ref[...] = i_ref[...].reshape(SLICE_PAGES, l, DPAD)[:, :, :dim]

    return pl.pallas_call(
        body,
        grid=(b // SLICE_PAGES,),
        in_specs=[pl.BlockSpec((SLICE_PAGES * l, DPAD), lambda i: (i, 0))],
        out_specs=pl.BlockSpec((SLICE_PAGES, l, dim), lambda i: (i, 0, 0)),
        out_shape=jax.ShapeDtypeStruct((b, l, dim), jnp.float32),
    )(o_pad)


def _gather_body(idx_hbm, out_hbm, w_hbm, idx_v, buf0, buf1, gsems, wsems,
                 wid, nchunks):
    base = pl.multiple_of(wid * nchunks * CHUNK, CHUNK)

    pltpu.sync_copy(idx_hbm.at[wid], idx_v)

    def step(i, _):
        c0 = pl.multiple_of(2 * i * CHUNK, CHUNK)
        c1 = pl.multiple_of((2 * i + 1) * CHUNK, CHUNK)
        g0 = pltpu.async_copy(w_hbm.at[idx_v.at[2 * i]], buf0, gsems[0])
        g1 = pltpu.async_copy(w_hbm.at[idx_v.at[2 * i + 1]], buf1, gsems[1])
        g0.wait()
        w0 = pltpu.async_copy(buf0, out_hbm.at[pl.ds(base + c0, CHUNK)],
                              wsems[0])
        g1.wait()
        w1 = pltpu.async_copy(buf1, out_hbm.at[pl.ds(base + c1, CHUNK)],
                              wsems[1])
        w0.wait()
        w1.wait()
        return 0

    lax.fori_loop(0, nchunks // 2, step, 0)


def _make_sc_gather(n_total):
    n_per_w = n_total // NW
    nchunks = n_per_w // CHUNK
    mesh = plsc.VectorSubcoreMesh(core_axis_name="c", subcore_axis_name="s")

    @functools.partial(
        pl.kernel,
        out_type=jax.ShapeDtypeStruct((n_total, DPAD), jnp.float32),
        mesh=mesh,
        scratch_types=[
            pltpu.VMEM((nchunks, CHUNK), jnp.int32),
            pltpu.VMEM((CHUNK, DPAD), jnp.float32),
            pltpu.VMEM((CHUNK, DPAD), jnp.float32),
            pltpu.SemaphoreType.DMA,
            pltpu.SemaphoreType.DMA,
            pltpu.SemaphoreType.DMA,
            pltpu.SemaphoreType.DMA,
        ],
    )
    def sc_gather(idx_hbm, w_hbm, out_hbm, idx_v, buf0, buf1, g0, g1, w0,
                  w1):
        wid = lax.axis_index("s") * NC + lax.axis_index("c")
        _gather_body(idx_hbm, out_hbm, w_hbm, idx_v, buf0, buf1,
                     (g0, g1), (w0, w1), wid, nchunks)

    return sc_gather


def kernel(target_word, synonym, antonym, W):
    b, l = target_word.shape
    dim = W.shape[1]
    n = b * l
    nchunks = n // NW // CHUNK
    w_pad = _pad_table(W, dim)

    fn = _make_sc_gather(n)
    outs = []
    for idx in (target_word, synonym, antonym):
        idx3 = idx.reshape(NW, nchunks, CHUNK).astype(jnp.int32)
        outs.append(_slice_out(fn(idx3, w_pad), b, l, dim))
    return tuple(outs)


# speedup vs baseline: 1.6461x; 1.6461x over previous
import functools

import jax
import jax.numpy as jnp
from jax import lax
from jax.experimental import pallas as pl
from jax.experimental.pallas import tpu as pltpu
from jax.experimental.pallas import tpu_sc as plsc

NC = 2
NS = 16
NW = NC * NS

CHUNK = 128
TILE = 128
DPAD = 384


def _gather_body(idx_hbm, out_hbm, w_hbm, wt_hbm, idx_v, buf0, buf1, gsems,
                 wsems, wid, nchunks):
    base = pl.multiple_of(wid * nchunks * CHUNK, CHUNK)

    pltpu.sync_copy(idx_hbm.at[wid], idx_v)

    def start_gathers(c, buf, sem):
        idx = idx_v.at[c]
        g0 = pltpu.async_copy(w_hbm.at[idx, pl.ds(0, TILE)],
                              buf.at[:, pl.ds(0, TILE)], sem)
        g1 = pltpu.async_copy(w_hbm.at[idx, pl.ds(TILE, TILE)],
                              buf.at[:, pl.ds(TILE, TILE)], sem)
        g2 = pltpu.async_copy(wt_hbm.at[idx],
                              buf.at[:, pl.ds(2 * TILE, TILE)], sem)
        return (g0, g1, g2)

    def step(i, _):
        c0 = pl.multiple_of(2 * i * CHUNK, CHUNK)
        c1 = pl.multiple_of((2 * i + 1) * CHUNK, CHUNK)
        ga = start_gathers(2 * i, buf0, gsems[0])
        gb = start_gathers(2 * i + 1, buf1, gsems[1])
        for g in ga:
            g.wait()
        w0 = pltpu.async_copy(buf0, out_hbm.at[pl.ds(base + c0, CHUNK)],
                              wsems[0])
        for g in gb:
            g.wait()
        w1 = pltpu.async_copy(buf1, out_hbm.at[pl.ds(base + c1, CHUNK)],
                              wsems[1])
        w0.wait()
        w1.wait()
        return 0

    lax.fori_loop(0, nchunks // 2, step, 0)


def _make_sc_gather(n_total, n_tensors):
    n_per_w = n_total // NW
    nchunks = n_per_w // CHUNK
    mesh = plsc.VectorSubcoreMesh(core_axis_name="c", subcore_axis_name="s")

    @functools.partial(
        pl.kernel,
        out_type=[jax.ShapeDtypeStruct((n_total, DPAD), jnp.float32)
                  for _ in range(n_tensors)],
        mesh=mesh,
        scratch_types=[
            pltpu.VMEM((nchunks, CHUNK), jnp.int32),
            pltpu.VMEM((CHUNK, DPAD), jnp.float32),
            pltpu.VMEM((CHUNK, DPAD), jnp.float32),
            pltpu.SemaphoreType.DMA,
            pltpu.SemaphoreType.DMA,
            pltpu.SemaphoreType.DMA,
            pltpu.SemaphoreType.DMA,
        ],
    )
    def sc_gather(*refs):
        idx_refs = refs[:n_tensors]
        w_hbm = refs[n_tensors]
        wt_hbm = refs[n_tensors + 1]
        out_refs = refs[n_tensors + 2:2 * n_tensors + 2]
        idx_v, buf0, buf1, g0, g1, w0, w1 = refs[2 * n_tensors + 2:]
        wid = lax.axis_index("s") * NC + lax.axis_index("c")
        for idx_hbm, out_hbm in zip(idx_refs, out_refs):
            _gather_body(idx_hbm, out_hbm, w_hbm, wt_hbm, idx_v, buf0, buf1,
                         (g0, g1), (w0, w1), wid, nchunks)

    return sc_gather


def kernel(target_word, synonym, antonym, W):
    b, l = target_word.shape
    dim = W.shape[1]
    n = b * l
    nchunks = n // NW // CHUNK
    w_tail = jnp.pad(W[:, 2 * TILE:], ((0, 0), (0, 3 * TILE - dim)))

    def prep(idx):
        return idx.reshape(NW, nchunks, CHUNK).astype(jnp.int32)

    fn = _make_sc_gather(n, 3)
    outs = fn(prep(target_word), prep(synonym), prep(antonym), W, w_tail)
    return tuple(o[:, :dim].reshape(b, l, dim) for o in outs)
